# initial kernel scaffold (unmeasured)
import jax
import jax.numpy as jnp
from jax import lax
from jax.experimental import pallas as pl
from jax.experimental.pallas import tpu as pltpu

T_LOC = 512
T = 1024
D = 1024
F = 2048
E_LOC = 4


def kernel(x, router, W1, W2):
    def body(x_ref, r_ref, w1_ref, w2_ref, out_ref,
             xrecv, rrecv, w1v, w2v, pout, precv,
             send_sems, recv_sems, copy_sems):
        my_x = lax.axis_index("x")
        my_y = lax.axis_index("y")
        nbr = (my_x, 1 - my_y)

        barrier = pltpu.get_barrier_semaphore()
        pl.semaphore_signal(barrier, inc=1, device_id=nbr,
                            device_id_type=pl.DeviceIdType.MESH)
        pl.semaphore_wait(barrier, 1)

        rx = pltpu.make_async_remote_copy(
            src_ref=x_ref, dst_ref=xrecv,
            send_sem=send_sems.at[0], recv_sem=recv_sems.at[0],
            device_id=nbr, device_id_type=pl.DeviceIdType.MESH)
        rr = pltpu.make_async_remote_copy(
            src_ref=r_ref, dst_ref=rrecv,
            send_sem=send_sems.at[1], recv_sem=recv_sems.at[1],
            device_id=nbr, device_id_type=pl.DeviceIdType.MESH)
        rx.start()
        rr.start()
        rx.wait()
        rr.wait()

        x_all = jnp.concatenate([x_ref[...], xrecv[...]], axis=0)
        r_full = jnp.concatenate([r_ref[...], rrecv[...]], axis=1)

        gates = jnp.dot(x_all, r_full, preferred_element_type=jnp.float32)
        m1 = jnp.max(gates, axis=1, keepdims=True)
        m2 = jnp.max(jnp.where(gates == m1, -jnp.inf, gates),
                     axis=1, keepdims=True)
        sel = gates >= m2
        wts = jnp.where(sel, jnp.exp(gates - m1), 0.0) / (1.0 + jnp.exp(m2 - m1))
        wloc = wts[:, :E_LOC]

        xb = x_all.astype(jnp.bfloat16)
        acc = jnp.zeros((T, D), jnp.float32)
        for e in range(E_LOC):
            c1 = pltpu.make_async_copy(w1_ref.at[e], w1v, copy_sems.at[0])
            c2 = pltpu.make_async_copy(w2_ref.at[e], w2v, copy_sems.at[1])
            c1.start()
            c2.start()
            c1.wait()
            c2.wait()
            h = jnp.dot(xb, w1v[...].astype(jnp.bfloat16),
                        preferred_element_type=jnp.float32)
            h = jnp.maximum(h, 0.0).astype(jnp.bfloat16)
            o = jnp.dot(h, w2v[...].astype(jnp.bfloat16),
                        preferred_element_type=jnp.float32)
            acc = acc + o * wloc[:, e:e + 1]
        pout[...] = acc

        rp = pltpu.make_async_remote_copy(
            src_ref=pout.at[pl.ds(T_LOC, T_LOC), :], dst_ref=precv,
            send_sem=send_sems.at[2], recv_sem=recv_sems.at[2],
            device_id=nbr, device_id_type=pl.DeviceIdType.MESH)
        rp.start()
        rp.wait()
        out_ref[...] = pout[pl.ds(0, T_LOC), :] + precv[...]

    return pl.pallas_call(
        body,
        out_shape=jax.ShapeDtypeStruct((T_LOC, D), jnp.float32),
        in_specs=[
            pl.BlockSpec(memory_space=pltpu.VMEM),
            pl.BlockSpec(memory_space=pltpu.VMEM),
            pl.BlockSpec(memory_space=pltpu.ANY),
            pl.BlockSpec(memory_space=pltpu.ANY),
        ],
        out_specs=pl.BlockSpec(memory_space=pltpu.VMEM),
        scratch_shapes=[
            pltpu.VMEM((T_LOC, D), jnp.float32),
            pltpu.VMEM((D, E_LOC), jnp.float32),
            pltpu.VMEM((D, F), jnp.float32),
            pltpu.VMEM((F, D), jnp.float32),
            pltpu.VMEM((T, D), jnp.float32),
            pltpu.VMEM((T_LOC, D), jnp.float32),
            pltpu.SemaphoreType.DMA((3,)),
            pltpu.SemaphoreType.DMA((3,)),
            pltpu.SemaphoreType.DMA((2,)),
        ],
        compiler_params=pltpu.CompilerParams(collective_id=0),
    )(x, router, W1, W2)


# baseline (device time: 129906 ns/iter reference)
import jax
import jax.numpy as jnp
from jax import lax
from jax.experimental import pallas as pl
from jax.experimental.pallas import tpu as pltpu

T_LOC = 512
T = 1024
D = 1024
F = 2048
E_LOC = 4


def kernel(x, router, W1, W2):
    def body(x_ref, r_ref, w1_ref, w2_ref, out_ref,
             xrecv, rrecv, w1v, w2v, pout, precv,
             send_sems, recv_sems, copy_sems):
        my_x = lax.axis_index("x")
        my_y = lax.axis_index("y")
        nbr = (my_x, 1 - my_y)

        barrier = pltpu.get_barrier_semaphore()
        pl.semaphore_signal(barrier, inc=1, device_id=nbr,
                            device_id_type=pl.DeviceIdType.MESH)
        pl.semaphore_wait(barrier, 1)

        rx = pltpu.make_async_remote_copy(
            src_ref=x_ref, dst_ref=xrecv,
            send_sem=send_sems.at[0], recv_sem=recv_sems.at[0],
            device_id=nbr, device_id_type=pl.DeviceIdType.MESH)
        rr = pltpu.make_async_remote_copy(
            src_ref=r_ref, dst_ref=rrecv,
            send_sem=send_sems.at[1], recv_sem=recv_sems.at[1],
            device_id=nbr, device_id_type=pl.DeviceIdType.MESH)
        rx.start()
        rr.start()
        rx.wait()
        rr.wait()

        x_all = jnp.concatenate([x_ref[...], xrecv[...]], axis=0)
        r_full = jnp.concatenate([r_ref[...], rrecv[...]], axis=1)

        xh = x_all.astype(jnp.bfloat16)
        xl = (x_all - xh.astype(jnp.float32)).astype(jnp.bfloat16)
        rh = r_full.astype(jnp.bfloat16)
        rl = (r_full - rh.astype(jnp.float32)).astype(jnp.bfloat16)
        gates = (jnp.dot(xh, rh, preferred_element_type=jnp.float32)
                 + jnp.dot(xl, rh, preferred_element_type=jnp.float32)
                 + jnp.dot(xh, rl, preferred_element_type=jnp.float32))
        m1 = jnp.max(gates, axis=1, keepdims=True)
        m2 = jnp.max(jnp.where(gates == m1, -jnp.inf, gates),
                     axis=1, keepdims=True)
        sel = gates >= m2
        wts = jnp.where(sel, jnp.exp(gates - m1), 0.0) / (1.0 + jnp.exp(m2 - m1))
        wloc = wts[:, :E_LOC]

        xb = xh
        acc = jnp.zeros((T, D), jnp.float32)
        for e in range(E_LOC):
            c1 = pltpu.make_async_copy(w1_ref.at[e], w1v, copy_sems.at[0])
            c2 = pltpu.make_async_copy(w2_ref.at[e], w2v, copy_sems.at[1])
            c1.start()
            c2.start()
            c1.wait()
            c2.wait()
            h = jnp.dot(xb, w1v[...].astype(jnp.bfloat16),
                        preferred_element_type=jnp.float32)
            h = jnp.maximum(h, 0.0).astype(jnp.bfloat16)
            o = jnp.dot(h, w2v[...].astype(jnp.bfloat16),
                        preferred_element_type=jnp.float32)
            acc = acc + o * wloc[:, e:e + 1]
        pout[...] = acc

        rp = pltpu.make_async_remote_copy(
            src_ref=pout.at[pl.ds(T_LOC, T_LOC), :], dst_ref=precv,
            send_sem=send_sems.at[2], recv_sem=recv_sems.at[2],
            device_id=nbr, device_id_type=pl.DeviceIdType.MESH)
        rp.start()
        rp.wait()
        out_ref[...] = pout[pl.ds(0, T_LOC), :] + precv[...]

    return pl.pallas_call(
        body,
        out_shape=jax.ShapeDtypeStruct((T_LOC, D), jnp.float32),
        in_specs=[
            pl.BlockSpec(memory_space=pltpu.VMEM),
            pl.BlockSpec(memory_space=pltpu.VMEM),
            pl.BlockSpec(memory_space=pl.ANY),
            pl.BlockSpec(memory_space=pl.ANY),
        ],
        out_specs=pl.BlockSpec(memory_space=pltpu.VMEM),
        scratch_shapes=[
            pltpu.VMEM((T_LOC, D), jnp.float32),
            pltpu.VMEM((D, E_LOC), jnp.float32),
            pltpu.VMEM((D, F), jnp.float32),
            pltpu.VMEM((F, D), jnp.float32),
            pltpu.VMEM((T, D), jnp.float32),
            pltpu.VMEM((T_LOC, D), jnp.float32),
            pltpu.SemaphoreType.DMA((3,)),
            pltpu.SemaphoreType.DMA((3,)),
            pltpu.SemaphoreType.DMA((2,)),
        ],
        compiler_params=pltpu.CompilerParams(
            collective_id=0,
            vmem_limit_bytes=100 * 1024 * 1024,
        ),
    )(x, router, W1, W2)


# device time: 73376 ns/iter; 1.7704x vs baseline; 1.7704x over previous
import jax
import jax.numpy as jnp
from jax import lax
from jax.experimental import pallas as pl
from jax.experimental.pallas import tpu as pltpu

T_LOC = 512
T = 1024
D = 1024
F = 2048
E_LOC = 4
BF = jnp.bfloat16
F32 = jnp.float32


def kernel(x, router, W1, W2):
    def body(x_ref, r_ref, w1_ref, w2_ref, out_ref,
             xsend, xrecv, rrecv, wsend, wrecv,
             w1v, w2v, psend, precv, ssend, srecv,
             send_sems, recv_sems, copy_sems):
        my_x = lax.axis_index("x")
        my_y = lax.axis_index("y")
        ynbr = (my_x, 1 - my_y)
        xnbr = (1 - my_x, my_y)

        barrier = pltpu.get_barrier_semaphore()
        for nbr in (ynbr, xnbr):
            pl.semaphore_signal(barrier, inc=1, device_id=nbr,
                                device_id_type=pl.DeviceIdType.MESH)
        pl.semaphore_wait(barrier, 2)

        xsend[...] = x_ref[...].astype(BF)
        rdma_x = pltpu.make_async_remote_copy(
            src_ref=xsend, dst_ref=xrecv,
            send_sem=send_sems.at[0], recv_sem=recv_sems.at[0],
            device_id=ynbr, device_id_type=pl.DeviceIdType.MESH)
        rdma_r = pltpu.make_async_remote_copy(
            src_ref=r_ref, dst_ref=rrecv,
            send_sem=send_sems.at[1], recv_sem=recv_sems.at[1],
            device_id=ynbr, device_id_type=pl.DeviceIdType.MESH)
        rdma_x.start()
        rdma_r.start()

        def fetch(j):
            @pl.when(my_x == 0)
            def _():
                pltpu.make_async_copy(w1_ref.at[j], w1v.at[j],
                                      copy_sems.at[j, 0]).start()
                pltpu.make_async_copy(w2_ref.at[j], w2v.at[j],
                                      copy_sems.at[j, 1]).start()

            @pl.when(my_x == 1)
            def _():
                pltpu.make_async_copy(w1_ref.at[2 + j], w1v.at[j],
                                      copy_sems.at[j, 0]).start()
                pltpu.make_async_copy(w2_ref.at[2 + j], w2v.at[j],
                                      copy_sems.at[j, 1]).start()

        def wait_fetch(j):
            pltpu.make_async_copy(w1_ref.at[j], w1v.at[j],
                                  copy_sems.at[j, 0]).wait()
            pltpu.make_async_copy(w2_ref.at[j], w2v.at[j],
                                  copy_sems.at[j, 1]).wait()

        fetch(0)
        fetch(1)

        rdma_r.wait()
        r_full = jnp.concatenate([r_ref[...], rrecv[...]], axis=1)
        xh = xsend[...]
        xl = (x_ref[...] - xh.astype(F32)).astype(BF)
        rh = r_full.astype(BF)
        rl = (r_full - rh.astype(F32)).astype(BF)
        gates = (jnp.dot(xh, rh, preferred_element_type=F32)
                 + jnp.dot(xl, rh, preferred_element_type=F32)
                 + jnp.dot(xh, rl, preferred_element_type=F32))

        m1 = jnp.max(gates, axis=1, keepdims=True)
        m2 = jnp.max(jnp.where(gates == m1, -jnp.inf, gates),
                     axis=1, keepdims=True)
        sel = gates >= m2
        wts = jnp.where(sel, jnp.exp(gates - m1), 0.0) / (1.0 + jnp.exp(m2 - m1))

        wsend[...] = wts[:, E_LOC:]
        rdma_w = pltpu.make_async_remote_copy(
            src_ref=wsend, dst_ref=wrecv,
            send_sem=send_sems.at[2], recv_sem=recv_sems.at[2],
            device_id=ynbr, device_id_type=pl.DeviceIdType.MESH)
        rdma_w.start()

        wmineA = jnp.where(my_x == 0, wts[:, 0:2], wts[:, 2:4])

        def ffn(rows_bf16, j):
            w1b = w1v[j].astype(BF)
            h = jnp.dot(rows_bf16, w1b, preferred_element_type=F32)
            h = jnp.maximum(h, 0.0).astype(BF)
            return jnp.dot(h, w2v[j].astype(BF), preferred_element_type=F32)

        wait_fetch(0)
        oA0 = ffn(xh, 0)

        rdma_x.wait()
        rdma_w.wait()
        wmineB = jnp.where(my_x == 0, wrecv[...][:, 0:2], wrecv[...][:, 2:4])
        xnb = xrecv[...]
        oB0 = ffn(xnb, 0)

        wait_fetch(1)
        oB1 = ffn(xnb, 1)
        accB = oB0 * wmineB[:, 0:1] + oB1 * wmineB[:, 1:2]
        psend[...] = accB.astype(BF)
        rdma_p = pltpu.make_async_remote_copy(
            src_ref=psend, dst_ref=precv,
            send_sem=send_sems.at[3], recv_sem=recv_sems.at[3],
            device_id=ynbr, device_id_type=pl.DeviceIdType.MESH)
        rdma_p.start()

        oA1 = ffn(xh, 1)
        accA = oA0 * wmineA[:, 0:1] + oA1 * wmineA[:, 1:2]

        rdma_p.wait()
        s = accA + precv[...].astype(F32)

        ssend[...] = s.astype(BF)
        rdma_s = pltpu.make_async_remote_copy(
            src_ref=ssend, dst_ref=srecv,
            send_sem=send_sems.at[4], recv_sem=recv_sems.at[4],
            device_id=xnbr, device_id_type=pl.DeviceIdType.MESH)
        rdma_s.start()
        rdma_s.wait()
        out_ref[...] = s + srecv[...].astype(F32)

    return pl.pallas_call(
        body,
        out_shape=jax.ShapeDtypeStruct((T_LOC, D), F32),
        in_specs=[
            pl.BlockSpec(memory_space=pltpu.VMEM),
            pl.BlockSpec(memory_space=pltpu.VMEM),
            pl.BlockSpec(memory_space=pl.ANY),
            pl.BlockSpec(memory_space=pl.ANY),
        ],
        out_specs=pl.BlockSpec(memory_space=pltpu.VMEM),
        scratch_shapes=[
            pltpu.VMEM((T_LOC, D), BF),
            pltpu.VMEM((T_LOC, D), BF),
            pltpu.VMEM((D, E_LOC), F32),
            pltpu.VMEM((T_LOC, E_LOC), F32),
            pltpu.VMEM((T_LOC, E_LOC), F32),
            pltpu.VMEM((2, D, F), F32),
            pltpu.VMEM((2, F, D), F32),
            pltpu.VMEM((T_LOC, D), BF),
            pltpu.VMEM((T_LOC, D), BF),
            pltpu.VMEM((T_LOC, D), BF),
            pltpu.VMEM((T_LOC, D), BF),
            pltpu.SemaphoreType.DMA((5,)),
            pltpu.SemaphoreType.DMA((5,)),
            pltpu.SemaphoreType.DMA((2, 2)),
        ],
        compiler_params=pltpu.CompilerParams(
            collective_id=0,
            vmem_limit_bytes=110 * 1024 * 1024,
        ),
    )(x, router, W1, W2)


# device time: 71818 ns/iter; 1.8088x vs baseline; 1.0217x over previous
import jax
import jax.numpy as jnp
from jax import lax
from jax.experimental import pallas as pl
from jax.experimental.pallas import tpu as pltpu

T_LOC = 512
T = 1024
D = 1024
F = 2048
E_LOC = 4
BF = jnp.bfloat16
F32 = jnp.float32


def kernel(x, router, W1, W2):
    def body(x_ref, r_ref, w1_ref, w2_ref, out_ref,
             xsend, xrecv, rrecv, wsend, wrecv,
             w1v, w2v, psend, precv, ssend, srecv,
             send_sems, recv_sems, copy_sems):
        my_x = lax.axis_index("x")
        my_y = lax.axis_index("y")
        ynbr = (my_x, 1 - my_y)
        xnbr = (1 - my_x, my_y)

        barrier = pltpu.get_barrier_semaphore()
        for nbr in (ynbr, xnbr):
            pl.semaphore_signal(barrier, inc=1, device_id=nbr,
                                device_id_type=pl.DeviceIdType.MESH)
        pl.semaphore_wait(barrier, 2)

        xsend[...] = x_ref[...].astype(BF)
        rdma_x = pltpu.make_async_remote_copy(
            src_ref=xsend, dst_ref=xrecv,
            send_sem=send_sems.at[0], recv_sem=recv_sems.at[0],
            device_id=ynbr, device_id_type=pl.DeviceIdType.MESH)
        rdma_r = pltpu.make_async_remote_copy(
            src_ref=r_ref, dst_ref=rrecv,
            send_sem=send_sems.at[1], recv_sem=recv_sems.at[1],
            device_id=ynbr, device_id_type=pl.DeviceIdType.MESH)
        rdma_x.start()
        rdma_r.start()

        def fetch(j):
            @pl.when(my_x == 0)
            def _():
                pltpu.make_async_copy(w1_ref.at[j], w1v.at[j],
                                      copy_sems.at[j, 0]).start()
                pltpu.make_async_copy(w2_ref.at[j], w2v.at[j],
                                      copy_sems.at[j, 1]).start()

            @pl.when(my_x == 1)
            def _():
                pltpu.make_async_copy(w1_ref.at[2 + j], w1v.at[j],
                                      copy_sems.at[j, 0]).start()
                pltpu.make_async_copy(w2_ref.at[2 + j], w2v.at[j],
                                      copy_sems.at[j, 1]).start()

        def wait_fetch(j):
            pltpu.make_async_copy(w1_ref.at[j], w1v.at[j],
                                  copy_sems.at[j, 0]).wait()
            pltpu.make_async_copy(w2_ref.at[j], w2v.at[j],
                                  copy_sems.at[j, 1]).wait()

        fetch(0)
        fetch(1)

        rdma_r.wait()
        r_full = jnp.concatenate([r_ref[...], rrecv[...]], axis=1)
        xh = xsend[...]
        xl = (x_ref[...] - xh.astype(F32)).astype(BF)
        rh = r_full.astype(BF)
        rl = (r_full - rh.astype(F32)).astype(BF)
        gates = (jnp.dot(xh, rh, preferred_element_type=F32)
                 + jnp.dot(xl, rh, preferred_element_type=F32)
                 + jnp.dot(xh, rl, preferred_element_type=F32))

        m1 = jnp.max(gates, axis=1, keepdims=True)
        m2 = jnp.max(jnp.where(gates == m1, -jnp.inf, gates),
                     axis=1, keepdims=True)
        sel = gates >= m2
        wts = jnp.where(sel, jnp.exp(gates - m1), 0.0) / (1.0 + jnp.exp(m2 - m1))

        wsend[...] = wts[:, E_LOC:]
        rdma_w = pltpu.make_async_remote_copy(
            src_ref=wsend, dst_ref=wrecv,
            send_sem=send_sems.at[2], recv_sem=recv_sems.at[2],
            device_id=ynbr, device_id_type=pl.DeviceIdType.MESH)
        rdma_w.start()

        wmineA = jnp.where(my_x == 0, wts[:, 0:2], wts[:, 2:4])

        def ffn(rows_bf16, w1b, w2b):
            h = jnp.dot(rows_bf16, w1b, preferred_element_type=F32)
            h = jnp.maximum(h, 0.0).astype(BF)
            return jnp.dot(h, w2b, preferred_element_type=F32)

        wait_fetch(0)
        w1b0 = w1v[0].astype(BF)
        w2b0 = w2v[0].astype(BF)
        oA0 = ffn(xh, w1b0, w2b0)

        rdma_x.wait()
        rdma_w.wait()
        wmineB = jnp.where(my_x == 0, wrecv[...][:, 0:2], wrecv[...][:, 2:4])
        xnb = xrecv[...]
        oB0 = ffn(xnb, w1b0, w2b0)

        wait_fetch(1)
        w1b1 = w1v[1].astype(BF)
        w2b1 = w2v[1].astype(BF)
        oB1 = ffn(xnb, w1b1, w2b1)
        accB = oB0 * wmineB[:, 0:1] + oB1 * wmineB[:, 1:2]
        psend[...] = accB.astype(BF)
        C = T_LOC // 2
        rp = []
        for c in range(2):
            r = pltpu.make_async_remote_copy(
                src_ref=psend.at[pl.ds(c * C, C), :],
                dst_ref=precv.at[pl.ds(c * C, C), :],
                send_sem=send_sems.at[3 + c], recv_sem=recv_sems.at[3 + c],
                device_id=ynbr, device_id_type=pl.DeviceIdType.MESH)
            r.start()
            rp.append(r)

        oA1 = ffn(xh, w1b1, w2b1)
        accA = oA0 * wmineA[:, 0:1] + oA1 * wmineA[:, 1:2]

        rs = []
        sc = []
        for c in range(2):
            rp[c].wait()
            s = accA[c * C:(c + 1) * C] + precv[pl.ds(c * C, C), :].astype(F32)
            sc.append(s)
            ssend[pl.ds(c * C, C), :] = s.astype(BF)
            r = pltpu.make_async_remote_copy(
                src_ref=ssend.at[pl.ds(c * C, C), :],
                dst_ref=srecv.at[pl.ds(c * C, C), :],
                send_sem=send_sems.at[5 + c], recv_sem=recv_sems.at[5 + c],
                device_id=xnbr, device_id_type=pl.DeviceIdType.MESH)
            r.start()
            rs.append(r)
        for c in range(2):
            rs[c].wait()
            out_ref[pl.ds(c * C, C), :] = sc[c] + srecv[pl.ds(c * C, C), :].astype(F32)

    return pl.pallas_call(
        body,
        out_shape=jax.ShapeDtypeStruct((T_LOC, D), F32),
        in_specs=[
            pl.BlockSpec(memory_space=pltpu.VMEM),
            pl.BlockSpec(memory_space=pltpu.VMEM),
            pl.BlockSpec(memory_space=pl.ANY),
            pl.BlockSpec(memory_space=pl.ANY),
        ],
        out_specs=pl.BlockSpec(memory_space=pltpu.VMEM),
        scratch_shapes=[
            pltpu.VMEM((T_LOC, D), BF),
            pltpu.VMEM((T_LOC, D), BF),
            pltpu.VMEM((D, E_LOC), F32),
            pltpu.VMEM((T_LOC, E_LOC), F32),
            pltpu.VMEM((T_LOC, E_LOC), F32),
            pltpu.VMEM((2, D, F), F32),
            pltpu.VMEM((2, F, D), F32),
            pltpu.VMEM((T_LOC, D), BF),
            pltpu.VMEM((T_LOC, D), BF),
            pltpu.VMEM((T_LOC, D), BF),
            pltpu.VMEM((T_LOC, D), BF),
            pltpu.SemaphoreType.DMA((7,)),
            pltpu.SemaphoreType.DMA((7,)),
            pltpu.SemaphoreType.DMA((2, 2)),
        ],
        compiler_params=pltpu.CompilerParams(
            collective_id=0,
            vmem_limit_bytes=110 * 1024 * 1024,
        ),
    )(x, router, W1, W2)


# device time: 58282 ns/iter; 2.2289x vs baseline; 1.2323x over previous
import jax
import jax.numpy as jnp
from jax import lax
from jax.experimental import pallas as pl
from jax.experimental.pallas import tpu as pltpu

T_LOC = 512
D = 1024
F = 2048
E_LOC = 4
Q = T_LOC // 2
H = 128
BF = jnp.bfloat16
F32 = jnp.float32


def kernel(x, router, W1, W2):
    def body(x_ref, r_ref, w1_ref, w2_ref, out_ref,
             xsend, xrecv, rrecv, wsend, wrecv, wA_ref,
             w1v, w2v, psend, precv, ssend, srecv,
             send_sems, recv_sems, copy_sems):
        my_x = lax.axis_index("x")
        my_y = lax.axis_index("y")
        ynbr = (my_x, 1 - my_y)
        xnbr = (1 - my_x, my_y)
        qoff = my_x * Q
        foff = (1 - my_x) * Q
        OFFS = [qoff, qoff + H, foff, foff + H]

        barrier = pltpu.get_barrier_semaphore()
        for nbr in (ynbr, xnbr):
            pl.semaphore_signal(barrier, inc=1, device_id=nbr,
                                device_id_type=pl.DeviceIdType.MESH)
        pl.semaphore_wait(barrier, 2)

        xsend[...] = x_ref[...].astype(BF)
        rdma_x = pltpu.make_async_remote_copy(
            src_ref=xsend.at[pl.ds(qoff, Q), :],
            dst_ref=xrecv.at[pl.ds(qoff, Q), :],
            send_sem=send_sems.at[0], recv_sem=recv_sems.at[0],
            device_id=ynbr, device_id_type=pl.DeviceIdType.MESH)
        rdma_r = pltpu.make_async_remote_copy(
            src_ref=r_ref, dst_ref=rrecv,
            send_sem=send_sems.at[1], recv_sem=recv_sems.at[1],
            device_id=ynbr, device_id_type=pl.DeviceIdType.MESH)
        rdma_x.start()
        rdma_r.start()

        def fetch(j):
            @pl.when(my_x == 0)
            def _():
                pltpu.make_async_copy(w1_ref.at[j], w1v.at[j],
                                      copy_sems.at[j, 0]).start()
                pltpu.make_async_copy(w2_ref.at[j], w2v.at[j],
                                      copy_sems.at[j, 1]).start()

            @pl.when(my_x == 1)
            def _():
                pltpu.make_async_copy(w1_ref.at[2 + j], w1v.at[j],
                                      copy_sems.at[j, 0]).start()
                pltpu.make_async_copy(w2_ref.at[2 + j], w2v.at[j],
                                      copy_sems.at[j, 1]).start()

        def wait_fetch(j):
            pltpu.make_async_copy(w1_ref.at[j], w1v.at[j],
                                  copy_sems.at[j, 0]).wait()
            pltpu.make_async_copy(w2_ref.at[j], w2v.at[j],
                                  copy_sems.at[j, 1]).wait()

        fetch(0)
        fetch(1)

        rdma_r.wait()
        r_full = jnp.concatenate([r_ref[...], rrecv[...]], axis=1)
        xh = xsend[...]
        xl = (x_ref[...] - xh.astype(F32)).astype(BF)
        rh = r_full.astype(BF)
        rl = (r_full - rh.astype(F32)).astype(BF)
        gates = (jnp.dot(xh, rh, preferred_element_type=F32)
                 + jnp.dot(xl, rh, preferred_element_type=F32)
                 + jnp.dot(xh, rl, preferred_element_type=F32))

        m1 = jnp.max(gates, axis=1, keepdims=True)
        m2 = jnp.max(jnp.where(gates == m1, -jnp.inf, gates),
                     axis=1, keepdims=True)
        sel = gates >= m2
        wts = jnp.where(sel, jnp.exp(gates - m1), 0.0) / (1.0 + jnp.exp(m2 - m1))

        wsend[...] = wts[:, E_LOC:]
        rdma_w = pltpu.make_async_remote_copy(
            src_ref=wsend, dst_ref=wrecv,
            send_sem=send_sems.at[2], recv_sem=recv_sems.at[2],
            device_id=ynbr, device_id_type=pl.DeviceIdType.MESH)
        rdma_w.start()
        wA_ref[...] = jnp.where(my_x == 0, wts[:, 0:2], wts[:, 2:4])

        rdma_x.wait()
        rdma_fwd = pltpu.make_async_remote_copy(
            src_ref=xrecv.at[pl.ds(qoff, Q), :],
            dst_ref=xrecv.at[pl.ds(qoff, Q), :],
            send_sem=send_sems.at[3], recv_sem=recv_sems.at[3],
            device_id=xnbr, device_id_type=pl.DeviceIdType.MESH)
        rdma_fwd.start()

        wait_fetch(0)
        w1b0 = w1v[0].astype(BF)
        w2b0 = w2v[0].astype(BF)
        wait_fetch(1)
        w1b1 = w1v[1].astype(BF)
        w2b1 = w2v[1].astype(BF)
        rdma_w.wait()

        def ffn(rows_bf16, w1b, w2b):
            h = jnp.dot(rows_bf16, w1b, preferred_element_type=F32)
            h = jnp.maximum(h, 0.0).astype(BF)
            return jnp.dot(h, w2b, preferred_element_type=F32)

        def bchunk(i, off):
            xc = xrecv[pl.ds(off, H), :]
            w4 = wrecv[pl.ds(off, H), :]
            wc = jnp.where(my_x == 0, w4[:, 0:2], w4[:, 2:4])
            accBc = (ffn(xc, w1b0, w2b0) * wc[:, 0:1]
                     + ffn(xc, w1b1, w2b1) * wc[:, 1:2])
            psend[pl.ds(off, H), :] = accBc.astype(BF)
            r = pltpu.make_async_remote_copy(
                src_ref=psend.at[pl.ds(off, H), :],
                dst_ref=precv.at[pl.ds(off, H), :],
                send_sem=send_sems.at[4 + i], recv_sem=recv_sems.at[4 + i],
                device_id=ynbr, device_id_type=pl.DeviceIdType.MESH)
            r.start()
            return r

        def achunk(off):
            xc = xsend[pl.ds(off, H), :]
            wc = wA_ref[pl.ds(off, H), :]
            return (ffn(xc, w1b0, w2b0) * wc[:, 0:1]
                    + ffn(xc, w1b1, w2b1) * wc[:, 1:2])

        rp = [bchunk(0, OFFS[0]), bchunk(1, OFFS[1])]
        accA0 = achunk(OFFS[0])
        rdma_fwd.wait()
        rp.append(bchunk(2, OFFS[2]))
        rp.append(bchunk(3, OFFS[3]))

        sc = []
        rs = []
        for i in range(4):
            accAc = accA0 if i == 0 else achunk(OFFS[i])
            rp[i].wait()
            s = accAc + precv[pl.ds(OFFS[i], H), :].astype(F32)
            sc.append(s)
            ssend[pl.ds(OFFS[i], H), :] = s.astype(BF)
            r = pltpu.make_async_remote_copy(
                src_ref=ssend.at[pl.ds(OFFS[i], H), :],
                dst_ref=srecv.at[pl.ds(OFFS[i], H), :],
                send_sem=send_sems.at[8 + i], recv_sem=recv_sems.at[8 + i],
                device_id=xnbr, device_id_type=pl.DeviceIdType.MESH)
            r.start()
            rs.append(r)

        for k, i in enumerate((2, 3, 0, 1)):
            rs[k].wait()
            out_ref[pl.ds(OFFS[i], H), :] = (
                sc[i] + srecv[pl.ds(OFFS[i], H), :].astype(F32))

    return pl.pallas_call(
        body,
        out_shape=jax.ShapeDtypeStruct((T_LOC, D), F32),
        in_specs=[
            pl.BlockSpec(memory_space=pltpu.VMEM),
            pl.BlockSpec(memory_space=pltpu.VMEM),
            pl.BlockSpec(memory_space=pl.ANY),
            pl.BlockSpec(memory_space=pl.ANY),
        ],
        out_specs=pl.BlockSpec(memory_space=pltpu.VMEM),
        scratch_shapes=[
            pltpu.VMEM((T_LOC, D), BF),
            pltpu.VMEM((T_LOC, D), BF),
            pltpu.VMEM((D, E_LOC), F32),
            pltpu.VMEM((T_LOC, E_LOC), F32),
            pltpu.VMEM((T_LOC, E_LOC), F32),
            pltpu.VMEM((T_LOC, 2), F32),
            pltpu.VMEM((2, D, F), F32),
            pltpu.VMEM((2, F, D), F32),
            pltpu.VMEM((T_LOC, D), BF),
            pltpu.VMEM((T_LOC, D), BF),
            pltpu.VMEM((T_LOC, D), BF),
            pltpu.VMEM((T_LOC, D), BF),
            pltpu.SemaphoreType.DMA((12,)),
            pltpu.SemaphoreType.DMA((12,)),
            pltpu.SemaphoreType.DMA((2, 2)),
        ],
        compiler_params=pltpu.CompilerParams(
            collective_id=0,
            vmem_limit_bytes=110 * 1024 * 1024,
        ),
    )(x, router, W1, W2)


# device time: 58262 ns/iter; 2.2297x vs baseline; 1.0003x over previous
import jax
import jax.numpy as jnp
from jax import lax
from jax.experimental import pallas as pl
from jax.experimental.pallas import tpu as pltpu

T_LOC = 512
D = 1024
F = 2048
E_LOC = 4
Q = T_LOC // 2
H = 128
BF = jnp.bfloat16
F32 = jnp.float32


def kernel(x, router, W1, W2):
    def body(x_ref, r_ref, w1_ref, w2_ref, out_ref,
             xsend, xrecv, rrecv, wsend, wrecv, wA_ref,
             w1v, w2v, psend, precv, ssend, srecv,
             send_sems, recv_sems, copy_sems):
        my_x = lax.axis_index("x")
        my_y = lax.axis_index("y")
        ynbr = (my_x, 1 - my_y)
        xnbr = (1 - my_x, my_y)
        qoff = my_x * Q
        foff = (1 - my_x) * Q
        OFFS = [qoff, qoff + H, foff, foff + H]

        barrier = pltpu.get_barrier_semaphore()
        for nbr in (ynbr, xnbr):
            pl.semaphore_signal(barrier, inc=1, device_id=nbr,
                                device_id_type=pl.DeviceIdType.MESH)
        pl.semaphore_wait(barrier, 2)

        xsend[...] = x_ref[...].astype(BF)
        rdma_x = pltpu.make_async_remote_copy(
            src_ref=xsend.at[pl.ds(qoff, Q), :],
            dst_ref=xrecv.at[pl.ds(qoff, Q), :],
            send_sem=send_sems.at[0], recv_sem=recv_sems.at[0],
            device_id=ynbr, device_id_type=pl.DeviceIdType.MESH)
        rdma_r = pltpu.make_async_remote_copy(
            src_ref=r_ref, dst_ref=rrecv,
            send_sem=send_sems.at[1], recv_sem=recv_sems.at[1],
            device_id=ynbr, device_id_type=pl.DeviceIdType.MESH)
        rdma_x.start()
        rdma_r.start()

        def fetch(j):
            @pl.when(my_x == 0)
            def _():
                pltpu.make_async_copy(w1_ref.at[j], w1v.at[j],
                                      copy_sems.at[j, 0]).start()
                pltpu.make_async_copy(w2_ref.at[j], w2v.at[j],
                                      copy_sems.at[j, 1]).start()

            @pl.when(my_x == 1)
            def _():
                pltpu.make_async_copy(w1_ref.at[2 + j], w1v.at[j],
                                      copy_sems.at[j, 0]).start()
                pltpu.make_async_copy(w2_ref.at[2 + j], w2v.at[j],
                                      copy_sems.at[j, 1]).start()

        def wait_fetch(j):
            pltpu.make_async_copy(w1_ref.at[j], w1v.at[j],
                                  copy_sems.at[j, 0]).wait()
            pltpu.make_async_copy(w2_ref.at[j], w2v.at[j],
                                  copy_sems.at[j, 1]).wait()

        fetch(0)
        fetch(1)

        rdma_r.wait()
        r_full = jnp.concatenate([r_ref[...], rrecv[...]], axis=1)
        xh = xsend[...]
        xl = (x_ref[...] - xh.astype(F32)).astype(BF)
        rh = r_full.astype(BF)
        rl = (r_full - rh.astype(F32)).astype(BF)
        gates = (jnp.dot(xh, rh, preferred_element_type=F32)
                 + jnp.dot(xl, rh, preferred_element_type=F32)
                 + jnp.dot(xh, rl, preferred_element_type=F32))

        m1 = jnp.max(gates, axis=1, keepdims=True)
        m2 = jnp.max(jnp.where(gates == m1, -jnp.inf, gates),
                     axis=1, keepdims=True)
        sel = gates >= m2
        wts = jnp.where(sel, jnp.exp(gates - m1), 0.0) / (1.0 + jnp.exp(m2 - m1))

        wsend[...] = wts[:, E_LOC:]
        rdma_w = pltpu.make_async_remote_copy(
            src_ref=wsend, dst_ref=wrecv,
            send_sem=send_sems.at[2], recv_sem=recv_sems.at[2],
            device_id=ynbr, device_id_type=pl.DeviceIdType.MESH)
        rdma_w.start()
        wA_ref[...] = jnp.where(my_x == 0, wts[:, 0:2], wts[:, 2:4])

        wait_fetch(0)
        w1b0 = w1v[0].astype(BF)
        w2b0 = w2v[0].astype(BF)
        wait_fetch(1)
        w1b1 = w1v[1].astype(BF)
        w2b1 = w2v[1].astype(BF)
        rdma_w.wait()

        rdma_x.wait()
        rdma_fwd = pltpu.make_async_remote_copy(
            src_ref=xrecv.at[pl.ds(qoff, Q), :],
            dst_ref=xrecv.at[pl.ds(qoff, Q), :],
            send_sem=send_sems.at[3], recv_sem=recv_sems.at[3],
            device_id=xnbr, device_id_type=pl.DeviceIdType.MESH)
        rdma_fwd.start()

        def ffn(rows_bf16, w1b, w2b):
            h = jnp.dot(rows_bf16, w1b, preferred_element_type=F32)
            h = jnp.maximum(h, 0.0).astype(BF)
            return jnp.dot(h, w2b, preferred_element_type=F32)

        def bchunk(i, off):
            xc = xrecv[pl.ds(off, H), :]
            w4 = wrecv[pl.ds(off, H), :]
            wc = jnp.where(my_x == 0, w4[:, 0:2], w4[:, 2:4])
            accBc = (ffn(xc, w1b0, w2b0) * wc[:, 0:1]
                     + ffn(xc, w1b1, w2b1) * wc[:, 1:2])
            psend[pl.ds(off, H), :] = accBc.astype(BF)
            r = pltpu.make_async_remote_copy(
                src_ref=psend.at[pl.ds(off, H), :],
                dst_ref=precv.at[pl.ds(off, H), :],
                send_sem=send_sems.at[4 + i], recv_sem=recv_sems.at[4 + i],
                device_id=ynbr, device_id_type=pl.DeviceIdType.MESH)
            r.start()
            return r

        def achunk(off):
            xc = xsend[pl.ds(off, H), :]
            wc = wA_ref[pl.ds(off, H), :]
            return (ffn(xc, w1b0, w2b0) * wc[:, 0:1]
                    + ffn(xc, w1b1, w2b1) * wc[:, 1:2])

        rp = [bchunk(0, OFFS[0]), bchunk(1, OFFS[1])]
        accA0 = achunk(OFFS[0])
        rdma_fwd.wait()
        rp.append(bchunk(2, OFFS[2]))
        rp.append(bchunk(3, OFFS[3]))

        sc = []
        rs = []
        for i in range(4):
            accAc = accA0 if i == 0 else achunk(OFFS[i])
            rp[i].wait()
            s = accAc + precv[pl.ds(OFFS[i], H), :].astype(F32)
            sc.append(s)
            ssend[pl.ds(OFFS[i], H), :] = s.astype(BF)
            r = pltpu.make_async_remote_copy(
                src_ref=ssend.at[pl.ds(OFFS[i], H), :],
                dst_ref=srecv.at[pl.ds(OFFS[i], H), :],
                send_sem=send_sems.at[8 + i], recv_sem=recv_sems.at[8 + i],
                device_id=xnbr, device_id_type=pl.DeviceIdType.MESH)
            r.start()
            rs.append(r)

        for k, i in enumerate((2, 3, 0, 1)):
            rs[k].wait()
            out_ref[pl.ds(OFFS[i], H), :] = (
                sc[i] + srecv[pl.ds(OFFS[i], H), :].astype(F32))

    return pl.pallas_call(
        body,
        out_shape=jax.ShapeDtypeStruct((T_LOC, D), F32),
        in_specs=[
            pl.BlockSpec(memory_space=pltpu.VMEM),
            pl.BlockSpec(memory_space=pltpu.VMEM),
            pl.BlockSpec(memory_space=pl.ANY),
            pl.BlockSpec(memory_space=pl.ANY),
        ],
        out_specs=pl.BlockSpec(memory_space=pltpu.VMEM),
        scratch_shapes=[
            pltpu.VMEM((T_LOC, D), BF),
            pltpu.VMEM((T_LOC, D), BF),
            pltpu.VMEM((D, E_LOC), F32),
            pltpu.VMEM((T_LOC, E_LOC), F32),
            pltpu.VMEM((T_LOC, E_LOC), F32),
            pltpu.VMEM((T_LOC, 2), F32),
            pltpu.VMEM((2, D, F), F32),
            pltpu.VMEM((2, F, D), F32),
            pltpu.VMEM((T_LOC, D), BF),
            pltpu.VMEM((T_LOC, D), BF),
            pltpu.VMEM((T_LOC, D), BF),
            pltpu.VMEM((T_LOC, D), BF),
            pltpu.SemaphoreType.DMA((12,)),
            pltpu.SemaphoreType.DMA((12,)),
            pltpu.SemaphoreType.DMA((2, 2)),
        ],
        compiler_params=pltpu.CompilerParams(
            collective_id=0,
            vmem_limit_bytes=110 * 1024 * 1024,
        ),
    )(x, router, W1, W2)
